# Initial kernel scaffold; baseline (speedup 1.0000x reference)
import jax
import jax.numpy as jnp
from jax.experimental import pallas as pl


def kernel(x, table, W1, W2, W3):
    raise NotImplementedError("write your pallas kernel here")


# speedup vs baseline: 81.7741x; 81.7741x over previous
import functools

import jax
import jax.numpy as jnp
from jax import lax
from jax.experimental import pallas as pl
from jax.experimental.pallas import tpu as pltpu
from jax.experimental.pallas import tpu_sc as plsc
import numpy as np

_N_LEVELS = 26
_F = 4
_LOG2_T = 17
_T = 1 << _LOG2_T
_BASE_RES = 32
_SCALE = 1.38
_N_POINTS = 262144
_D_ENC = _N_LEVELS * _F
_HIDDEN = 128
_OUT_DIM = 3
_P1 = np.uint32(2654435761)
_P2 = np.uint32(805459861)

_NC = 2
_NS = 16
_NW = _NC * _NS
_PPW = _N_POINTS // _NW
_C = 128
_NCHUNK = _PPW // _C

_RES = [int(np.floor(_BASE_RES * (_SCALE ** l))) for l in range(_N_LEVELS)]
_DENSE = [l for l in range(_N_LEVELS) if (_RES[l] + 1) ** 3 <= _T]
_HASH_LO = len(_DENSE)


def _iota16():
    return lax.broadcasted_iota(jnp.int32, (16,), 0)


def _enc_body(xf_hbm, tbl_hbm, resf_hbm, enc_hbm,
              xf, fx, fy, fz, idxb, rows, outc, cres, gsem):
    cid = lax.axis_index("c")
    sid = lax.axis_index("s")
    wid = sid * _NC + cid
    base0 = wid * _PPW

    pltpu.sync_copy(resf_hbm, cres)
    iota = _iota16()
    i2c = iota & 3

    def calc_level(resf_vec, mul1, mul2, use_xor, off_vec):
        def vb(k, _):
            p3 = (k * 16 + iota) * 3
            xv = plsc.load_gather(xf, [p3])
            yv = plsc.load_gather(xf, [p3 + 1])
            zv = plsc.load_gather(xf, [p3 + 2])
            posx = xv * resf_vec
            posy = yv * resf_vec
            posz = zv * resf_vec
            c0x = posx.astype(jnp.int32)
            c0y = posy.astype(jnp.int32)
            c0z = posz.astype(jnp.int32)
            sl = pl.ds(k * 16, 16)
            fx[sl] = posx - c0x.astype(jnp.float32)
            fy[sl] = posy - c0y.astype(jnp.float32)
            fz[sl] = posz - c0z.astype(jnp.float32)
            ax = (c0x, c0x + 1)
            b0 = c0y * mul1
            by = (b0, b0 + mul1)
            c0 = c0z * mul2
            cz = (c0, c0 + mul2)
            for j in range(8):
                a = ax[(j >> 2) & 1]
                b = by[(j >> 1) & 1]
                c = cz[j & 1]
                h = (a ^ b ^ c) if use_xor else (a + b + c)
                idxb[j, sl] = (h & (_T - 1)) + off_vec
            return 0
        lax.fori_loop(0, _C // 16, vb, 0)

    def interp_level(col0_vec):
        def vb(k, _):
            e = k * 16 + iota
            i1 = e >> 2
            fxv = plsc.load_gather(fx, [i1])
            fyv = plsc.load_gather(fy, [i1])
            fzv = plsc.load_gather(fz, [i1])
            r = [plsc.load_gather(rows, [jnp.full((16,), j, jnp.int32), i1, i2c])
                 for j in range(8)]
            m00 = r[0] + fzv * (r[1] - r[0])
            m01 = r[2] + fzv * (r[3] - r[2])
            m10 = r[4] + fzv * (r[5] - r[4])
            m11 = r[6] + fzv * (r[7] - r[6])
            my0 = m00 + fyv * (m01 - m00)
            my1 = m10 + fyv * (m11 - m10)
            val = my0 + fxv * (my1 - my0)
            plsc.store_scatter(outc, [i1, col0_vec + i2c], val)
            return 0
        lax.fori_loop(0, (_C * 4) // 16, vb, 0)

    def gather_level():
        cps = [pltpu.make_async_copy(tbl_hbm.at[idxb.at[j]], rows.at[j], gsem)
               for j in range(8)]
        for cp in cps:
            cp.start()
        for cp in cps:
            cp.wait()

    def chunk_body(ci, _):
        base = base0 + ci * _C
        pltpu.sync_copy(xf_hbm.at[pl.ds(base * 3, 3 * _C)], xf)
        for l in range(_HASH_LO):
            s = _RES[l] + 1
            calc_level(jnp.full((16,), float(_RES[l]), jnp.float32),
                       jnp.int32(s), jnp.int32(s * s), False,
                       jnp.full((16,), l * _T, jnp.int32))
            gather_level()
            interp_level(jnp.full((16,), l * 4, jnp.int32))

        def lvl_body(l, _):
            resf_vec = cres[pl.ds(l * 16, 16)]
            lv = jnp.full((16,), 1, jnp.int32) * l
            calc_level(resf_vec, jnp.int32(_P1), jnp.int32(_P2), True,
                       lv << _LOG2_T)
            gather_level()
            interp_level(lv * 4)
            return 0
        lax.fori_loop(_HASH_LO, _N_LEVELS, lvl_body, 0)

        pltpu.sync_copy(outc, enc_hbm.at[pl.ds(base, _C)])
        return 0

    lax.fori_loop(0, _NCHUNK, chunk_body, 0)


@jax.jit
def _encode(xf, tbl, resf):
    mesh = plsc.VectorSubcoreMesh(core_axis_name="c", subcore_axis_name="s",
                                  num_cores=_NC, num_subcores=_NS)
    f = functools.partial(
        pl.kernel,
        out_type=jax.ShapeDtypeStruct((_N_POINTS, _D_ENC), jnp.float32),
        mesh=mesh,
        scratch_types=[
            pltpu.VMEM((3 * _C,), jnp.float32),
            pltpu.VMEM((_C,), jnp.float32),
            pltpu.VMEM((_C,), jnp.float32),
            pltpu.VMEM((_C,), jnp.float32),
            pltpu.VMEM((8, _C), jnp.int32),
            pltpu.VMEM((8, _C, 8), jnp.float32),
            pltpu.VMEM((_C, _D_ENC), jnp.float32),
            pltpu.VMEM((_N_LEVELS * 16,), jnp.float32),
            pltpu.SemaphoreType.DMA,
        ],
        compiler_params=pltpu.CompilerParams(needs_layout_passes=False,
                                             use_tc_tiling_on_sc=False),
    )(_enc_body)
    return f(xf, tbl, resf)


def _mlp_body(enc_ref, w1_ref, w2_ref, w3_ref, o_ref):
    h = jnp.dot(enc_ref[...], w1_ref[...], preferred_element_type=jnp.float32)
    h = jnp.maximum(h, 0.0)
    h = jnp.dot(h, w2_ref[...], preferred_element_type=jnp.float32)
    h = jnp.maximum(h, 0.0)
    o_ref[...] = jnp.dot(h, w3_ref[...], preferred_element_type=jnp.float32)


@jax.jit
def _mlp(enc, W1, W2, W3):
    bn = 2048
    return pl.pallas_call(
        _mlp_body,
        grid=(_N_POINTS // bn,),
        in_specs=[
            pl.BlockSpec((bn, _D_ENC), lambda i: (i, 0)),
            pl.BlockSpec((_D_ENC, _HIDDEN), lambda i: (0, 0)),
            pl.BlockSpec((_HIDDEN, _HIDDEN), lambda i: (0, 0)),
            pl.BlockSpec((_HIDDEN, _OUT_DIM), lambda i: (0, 0)),
        ],
        out_specs=pl.BlockSpec((bn, _OUT_DIM), lambda i: (i, 0)),
        out_shape=jax.ShapeDtypeStruct((_N_POINTS, _OUT_DIM), jnp.float32),
    )(enc, W1, W2, W3)


_RESF_TAB = np.repeat(np.array(_RES, np.float32)[:, None], 16, axis=1).reshape(-1)


def kernel(x, table, W1, W2, W3):
    xf = x.reshape(-1)
    tbl = jnp.pad(table.reshape(_N_LEVELS * _T, _F), ((0, 0), (0, 8 - _F)))
    enc = _encode(xf, tbl, jnp.asarray(_RESF_TAB))
    return _mlp(enc, W1, W2, W3)


# speedup vs baseline: 110.2246x; 1.3479x over previous
import functools

import jax
import jax.numpy as jnp
from jax import lax
from jax.experimental import pallas as pl
from jax.experimental.pallas import tpu as pltpu
from jax.experimental.pallas import tpu_sc as plsc
import numpy as np

_N_LEVELS = 26
_F = 4
_LOG2_T = 17
_T = 1 << _LOG2_T
_BASE_RES = 32
_SCALE = 1.38
_N_POINTS = 262144
_D_ENC = _N_LEVELS * _F
_HIDDEN = 128
_OUT_DIM = 3
_P1 = np.uint32(2654435761)
_P2 = np.uint32(805459861)

_NC = 2
_NS = 16
_NW = _NC * _NS
_PPW = _N_POINTS // _NW
_C = 128
_NCHUNK = _PPW // _C

_RES = [int(np.floor(_BASE_RES * (_SCALE ** l))) for l in range(_N_LEVELS)]
_DENSE = [l for l in range(_N_LEVELS) if (_RES[l] + 1) ** 3 <= _T]
_HASH_LO = len(_DENSE)

_RESF_TAB = np.repeat(np.array(_RES, np.float32)[:, None], 16, axis=1).reshape(-1)
_M1 = np.array([(_RES[l] + 1) if l < _HASH_LO else _P1 for l in range(_N_LEVELS)],
               np.uint32).astype(np.int64).astype(np.int32)
_M2 = np.array([(_RES[l] + 1) ** 2 if l < _HASH_LO else _P2 for l in range(_N_LEVELS)],
               np.uint32).astype(np.int64).astype(np.int32)
_M1_TAB = np.repeat(_M1[:, None], 16, axis=1).reshape(-1)
_M2_TAB = np.repeat(_M2[:, None], 16, axis=1).reshape(-1)


def _iota16():
    return lax.broadcasted_iota(jnp.int32, (16,), 0)


def _enc_body(xf_hbm, tbl_hbm, resf_hbm, m1_hbm, m2_hbm, enc_hbm,
              xf, xc, outc, cres, cm1, cm2,
              fxA, fyA, fzA, idxA, poA, rowsA, semA,
              fxB, fyB, fzB, idxB, poB, rowsB, semB):
    cid = lax.axis_index("c")
    sid = lax.axis_index("s")
    wid = sid * _NC + cid
    base0 = wid * _PPW

    pltpu.sync_copy(resf_hbm, cres)
    pltpu.sync_copy(m1_hbm, cm1)
    pltpu.sync_copy(m2_hbm, cm2)
    iota = _iota16()
    i2c = iota & 3

    def calc_level(l, fx, fy, fz, idxb, pob):
        l16 = pl.ds(l * 16, 16)
        resf_vec = cres[l16]
        mul1 = cm1[l16]
        mul2 = cm2[l16]
        dense = jnp.full((16,), 1, jnp.int32) * l < _HASH_LO
        off_vec = (jnp.full((16,), 1, jnp.int32) * l) << _LOG2_T

        def vb(k, _):
            sl = pl.ds(k * 16, 16)
            xv = xc[pl.ds(k * 16, 16)]
            yv = xc[pl.ds(_C + k * 16, 16)]
            zv = xc[pl.ds(2 * _C + k * 16, 16)]
            posx = xv * resf_vec
            posy = yv * resf_vec
            posz = zv * resf_vec
            c0x = posx.astype(jnp.int32)
            c0y = posy.astype(jnp.int32)
            c0z = posz.astype(jnp.int32)
            fx[sl] = posx - c0x.astype(jnp.float32)
            fy[sl] = posy - c0y.astype(jnp.float32)
            fz[sl] = posz - c0z.astype(jnp.float32)
            ax = (c0x, c0x + 1)
            b0 = c0y * mul1
            by = (b0, b0 + mul1)
            c0 = c0z * mul2
            cz = (c0, c0 + mul2)
            for j in range(8):
                a = ax[(j >> 2) & 1]
                b = by[(j >> 1) & 1]
                c = cz[j & 1]
                h = jnp.where(dense, a + b + c, a ^ b ^ c)
                idx = (h & (_T - 1)) + off_vec
                idxb[j, sl] = idx >> 1
                pob[j, sl] = (idx & 1) << 2
            return 0
        lax.fori_loop(0, _C // 16, vb, 0)

    def fire(idxb, rows, sem):
        for j in range(8):
            pltpu.make_async_copy(tbl_hbm.at[idxb.at[j]], rows.at[j], sem).start()

    def drain(idxb, rows, sem):
        for j in range(8):
            pltpu.make_async_copy(tbl_hbm.at[idxb.at[j]], rows.at[j], sem).wait()

    def interp_level(l, fx, fy, fz, rows, pob):
        col0_vec = (jnp.full((16,), 1, jnp.int32) * l) * 4 + i2c

        def vb(k, _):
            e = k * 16 + iota
            i1 = e >> 2
            fxv = plsc.load_gather(fx, [i1])
            fyv = plsc.load_gather(fy, [i1])
            fzv = plsc.load_gather(fz, [i1])
            jvs = [jnp.full((16,), j, jnp.int32) for j in range(8)]
            r = [plsc.load_gather(rows,
                                  [jvs[j], i1,
                                   plsc.load_gather(pob, [jvs[j], i1]) + i2c])
                 for j in range(8)]
            m00 = r[0] + fzv * (r[1] - r[0])
            m01 = r[2] + fzv * (r[3] - r[2])
            m10 = r[4] + fzv * (r[5] - r[4])
            m11 = r[6] + fzv * (r[7] - r[6])
            my0 = m00 + fyv * (m01 - m00)
            my1 = m10 + fyv * (m11 - m10)
            val = my0 + fxv * (my1 - my0)
            plsc.store_scatter(outc, [(i1 << 7) + col0_vec], val)
            return 0
        lax.fori_loop(0, (_C * 4) // 16, vb, 0)

    fzero = jnp.zeros((16,), jnp.float32)

    def zb(k, _):
        outc[pl.ds(k * 16, 16)] = fzero
        return 0
    lax.fori_loop(0, (_C * 128) // 16, zb, 0)

    def chunk_body(ci, _):
        base = base0 + ci * _C
        pltpu.sync_copy(xf_hbm.at[pl.ds(base * 3, 3 * _C)], xf)

        def xb(k, _):
            p = k * 16 + iota
            for d in range(3):
                v = plsc.load_gather(xf, [p * 3 + d])
                xc[pl.ds(d * _C + k * 16, 16)] = v
            return 0
        lax.fori_loop(0, _C // 16, xb, 0)

        calc_level(0, fxA, fyA, fzA, idxA, poA)
        fire(idxA, rowsA, semA)

        def pair(l):
            calc_level(l + 1, fxB, fyB, fzB, idxB, poB)
            fire(idxB, rowsB, semB)
            drain(idxA, rowsA, semA)
            interp_level(l, fxA, fyA, fzA, rowsA, poA)

            @pl.when(l + 2 < _N_LEVELS)
            def _():
                calc_level(l + 2, fxA, fyA, fzA, idxA, poA)
                fire(idxA, rowsA, semA)
            drain(idxB, rowsB, semB)
            interp_level(l + 1, fxB, fyB, fzB, rowsB, poB)
        pl.loop(0, _N_LEVELS, step=2)(pair)

        pltpu.sync_copy(outc, enc_hbm.at[pl.ds(base * 128, _C * 128)])
        return 0

    lax.fori_loop(0, _NCHUNK, chunk_body, 0)


@jax.jit
def _encode(xf, tbl, resf, m1t, m2t):
    mesh = plsc.VectorSubcoreMesh(core_axis_name="c", subcore_axis_name="s",
                                  num_cores=_NC, num_subcores=_NS)
    dbuf = [
        pltpu.VMEM((_C,), jnp.float32),
        pltpu.VMEM((_C,), jnp.float32),
        pltpu.VMEM((_C,), jnp.float32),
        pltpu.VMEM((8, _C), jnp.int32),
        pltpu.VMEM((8, _C), jnp.int32),
        pltpu.VMEM((8, _C, 8), jnp.float32),
        pltpu.SemaphoreType.DMA,
    ]
    f = pl.kernel(
        _enc_body,
        out_type=jax.ShapeDtypeStruct((_N_POINTS * 128,), jnp.float32),
        mesh=mesh,
        scratch_types=[
            pltpu.VMEM((3 * _C,), jnp.float32),
            pltpu.VMEM((3 * _C,), jnp.float32),
            pltpu.VMEM((_C * 128,), jnp.float32),
            pltpu.VMEM((_N_LEVELS * 16,), jnp.float32),
            pltpu.VMEM((_N_LEVELS * 16,), jnp.int32),
            pltpu.VMEM((_N_LEVELS * 16,), jnp.int32),
        ] + dbuf + dbuf,
        compiler_params=pltpu.CompilerParams(needs_layout_passes=False,
                                             use_tc_tiling_on_sc=False),
    )
    return f(xf, tbl, resf, m1t, m2t)


def _mlp_body(enc_ref, w1_ref, w2_ref, w3_ref, o_ref):
    h = jnp.dot(enc_ref[...], w1_ref[...], preferred_element_type=jnp.float32)
    h = jnp.maximum(h, 0.0)
    h = jnp.dot(h, w2_ref[...], preferred_element_type=jnp.float32)
    h = jnp.maximum(h, 0.0)
    o_ref[...] = jnp.dot(h, w3_ref[...], preferred_element_type=jnp.float32)


@jax.jit
def _mlp(enc, W1, W2, W3):
    bn = 2048
    return pl.pallas_call(
        _mlp_body,
        grid=(_N_POINTS // bn,),
        in_specs=[
            pl.BlockSpec((bn, 128), lambda i: (i, 0)),
            pl.BlockSpec((128, _HIDDEN), lambda i: (0, 0)),
            pl.BlockSpec((_HIDDEN, _HIDDEN), lambda i: (0, 0)),
            pl.BlockSpec((_HIDDEN, _OUT_DIM), lambda i: (0, 0)),
        ],
        out_specs=pl.BlockSpec((bn, _OUT_DIM), lambda i: (i, 0)),
        out_shape=jax.ShapeDtypeStruct((_N_POINTS, _OUT_DIM), jnp.float32),
    )(enc, W1, W2, W3)


def kernel(x, table, W1, W2, W3):
    xf = x.reshape(-1)
    tbl = table.reshape(_N_LEVELS * _T // 2, 2 * _F)
    enc = _encode(xf, tbl, jnp.asarray(_RESF_TAB),
                  jnp.asarray(_M1_TAB), jnp.asarray(_M2_TAB))
    w1p = jnp.pad(W1, ((0, 128 - _D_ENC), (0, 0)))
    return _mlp(enc.reshape(_N_POINTS, 128), w1p, W2, W3)


# speedup vs baseline: 234.8750x; 2.1309x over previous
import functools

import jax
import jax.numpy as jnp
from jax import lax
from jax.experimental import pallas as pl
from jax.experimental.pallas import tpu as pltpu
from jax.experimental.pallas import tpu_sc as plsc
import numpy as np

_N_LEVELS = 26
_F = 4
_LOG2_T = 17
_T = 1 << _LOG2_T
_BASE_RES = 32
_SCALE = 1.38
_N_POINTS = 262144
_D_ENC = _N_LEVELS * _F
_HIDDEN = 128
_OUT_DIM = 3
_P1 = np.uint32(2654435761)
_P2 = np.uint32(805459861)

_NC = 2
_NS = 16
_NW = _NC * _NS
_PPW = _N_POINTS // _NW
_C = 128
_NCHUNK = _PPW // _C

_RES = [int(np.floor(_BASE_RES * (_SCALE ** l))) for l in range(_N_LEVELS)]
_DENSE = [l for l in range(_N_LEVELS) if (_RES[l] + 1) ** 3 <= _T]
_HASH_LO = len(_DENSE)

_RESF_TAB = np.repeat(np.array(_RES, np.float32)[:, None], 16, axis=1).reshape(-1)
_M1 = np.array([(_RES[l] + 1) if l < _HASH_LO else _P1 for l in range(_N_LEVELS)],
               np.uint32).astype(np.int64).astype(np.int32)
_M2 = np.array([(_RES[l] + 1) ** 2 if l < _HASH_LO else _P2 for l in range(_N_LEVELS)],
               np.uint32).astype(np.int64).astype(np.int32)
_M1_TAB = np.repeat(_M1[:, None], 16, axis=1).reshape(-1)
_M2_TAB = np.repeat(_M2[:, None], 16, axis=1).reshape(-1)


def _iota16():
    return lax.broadcasted_iota(jnp.int32, (16,), 0)


_NT = _N_LEVELS * _T // 128
_TPS = _NT // _NS
_K = 8


def _enc_body(xf_hbm, tbl_hbm, resf_hbm, m1_hbm, m2_hbm, enc_hbm,
              xf, xc, outc, cres, cm1, cm2, bufA, bufB, tblq,
              fxA, fyA, fzA, idxA, poA, rowsA, semA,
              fxB, fyB, fzB, idxB, poB, rowsB, semB):
    cid = lax.axis_index("c")
    sid = lax.axis_index("s")
    wid = sid * _NC + cid
    base0 = wid * _PPW

    pltpu.sync_copy(resf_hbm, cres)
    pltpu.sync_copy(m1_hbm, cm1)
    pltpu.sync_copy(m2_hbm, cm2)
    iota = _iota16()
    i2c = iota & 3

    iq = iota >> 2
    ic = (iota & 3) << 2

    def tb(ti, _):
        t0 = sid * _TPS + ti * _K
        pltpu.sync_copy(tbl_hbm.at[pl.ds(t0 * 4, _K * 4)], bufA)

        def gb(g, _):
            tile = g >> 3
            grp = g & 7
            rowv = jnp.full((16,), 1, jnp.int32) * (tile * 32 + grp * 4) + iq
            ev = grp * 16 + iota
            for f in range(4):
                v = plsc.load_gather(
                    bufA, [jnp.full((16,), 1, jnp.int32) * (tile * 4 + f), ev])
                plsc.store_scatter(bufB, [rowv, ic + f], v)
            return 0
        lax.fori_loop(0, _K * 8, gb, 0)
        pltpu.sync_copy(bufB, tblq.at[pl.ds(t0 * 32, _K * 32)])
        return 0
    lax.fori_loop(0, _TPS // _K, tb, 0)
    plsc.subcore_barrier()

    def calc_level(l, fx, fy, fz, idxb, pob):
        l16 = pl.ds(l * 16, 16)
        resf_vec = cres[l16]
        mul1 = cm1[l16]
        mul2 = cm2[l16]
        dense = jnp.full((16,), 1, jnp.int32) * l < _HASH_LO
        off_vec = (jnp.full((16,), 1, jnp.int32) * l) << _LOG2_T

        def vb(k, _):
            sl = pl.ds(k * 16, 16)
            xv = xc[pl.ds(k * 16, 16)]
            yv = xc[pl.ds(_C + k * 16, 16)]
            zv = xc[pl.ds(2 * _C + k * 16, 16)]
            posx = xv * resf_vec
            posy = yv * resf_vec
            posz = zv * resf_vec
            c0x = posx.astype(jnp.int32)
            c0y = posy.astype(jnp.int32)
            c0z = posz.astype(jnp.int32)
            fx[sl] = posx - c0x.astype(jnp.float32)
            fy[sl] = posy - c0y.astype(jnp.float32)
            fz[sl] = posz - c0z.astype(jnp.float32)
            ax = (c0x, c0x + 1)
            b0 = c0y * mul1
            by = (b0, b0 + mul1)
            c0 = c0z * mul2
            cz = (c0, c0 + mul2)
            for j in range(8):
                a = ax[(j >> 2) & 1]
                b = by[(j >> 1) & 1]
                c = cz[j & 1]
                h = jnp.where(dense, a + b + c, a ^ b ^ c)
                idx = (h & (_T - 1)) + off_vec
                idxb[j, sl] = idx >> 2
                pob[j, sl] = (idx & 3) << 2
            return 0
        lax.fori_loop(0, _C // 16, vb, 0)

    def fire(idxb, rows, sem):
        for j in range(8):
            pltpu.make_async_copy(tblq.at[idxb.at[j]], rows.at[j], sem).start()

    def drain(idxb, rows, sem):
        for j in range(8):
            pltpu.make_async_copy(tblq.at[idxb.at[j]], rows.at[j], sem).wait()

    def interp_level(l, fx, fy, fz, rows, pob):
        col0_vec = (jnp.full((16,), 1, jnp.int32) * l) * 4 + i2c

        def vb(k, _):
            e = k * 16 + iota
            i1 = e >> 2
            fxv = plsc.load_gather(fx, [i1])
            fyv = plsc.load_gather(fy, [i1])
            fzv = plsc.load_gather(fz, [i1])
            jvs = [jnp.full((16,), j, jnp.int32) for j in range(8)]
            r = [plsc.load_gather(rows,
                                  [jvs[j], i1,
                                   plsc.load_gather(pob, [jvs[j], i1]) + i2c])
                 for j in range(8)]
            m00 = r[0] + fzv * (r[1] - r[0])
            m01 = r[2] + fzv * (r[3] - r[2])
            m10 = r[4] + fzv * (r[5] - r[4])
            m11 = r[6] + fzv * (r[7] - r[6])
            my0 = m00 + fyv * (m01 - m00)
            my1 = m10 + fyv * (m11 - m10)
            val = my0 + fxv * (my1 - my0)
            plsc.store_scatter(outc, [(i1 << 7) + col0_vec], val)
            return 0
        lax.fori_loop(0, (_C * 4) // 16, vb, 0)

    fzero = jnp.zeros((16,), jnp.float32)

    def zb(k, _):
        outc[pl.ds(k * 16, 16)] = fzero
        return 0
    lax.fori_loop(0, (_C * 128) // 16, zb, 0)

    def chunk_body(ci, _):
        base = base0 + ci * _C
        pltpu.sync_copy(xf_hbm.at[pl.ds(base * 3, 3 * _C)], xf)

        def xb(k, _):
            p = k * 16 + iota
            for d in range(3):
                v = plsc.load_gather(xf, [p * 3 + d])
                xc[pl.ds(d * _C + k * 16, 16)] = v
            return 0
        lax.fori_loop(0, _C // 16, xb, 0)

        calc_level(0, fxA, fyA, fzA, idxA, poA)
        fire(idxA, rowsA, semA)

        def pair(l):
            calc_level(l + 1, fxB, fyB, fzB, idxB, poB)
            fire(idxB, rowsB, semB)
            drain(idxA, rowsA, semA)
            interp_level(l, fxA, fyA, fzA, rowsA, poA)

            @pl.when(l + 2 < _N_LEVELS)
            def _():
                calc_level(l + 2, fxA, fyA, fzA, idxA, poA)
                fire(idxA, rowsA, semA)
            drain(idxB, rowsB, semB)
            interp_level(l + 1, fxB, fyB, fzB, rowsB, poB)
        pl.loop(0, _N_LEVELS, step=2)(pair)

        pltpu.sync_copy(outc, enc_hbm.at[pl.ds(base * 128, _C * 128)])
        return 0

    lax.fori_loop(0, _NCHUNK, chunk_body, 0)


@jax.jit
def _encode(xf, tbl, resf, m1t, m2t):
    mesh = plsc.VectorSubcoreMesh(core_axis_name="c", subcore_axis_name="s",
                                  num_cores=_NC, num_subcores=_NS)
    dbuf = [
        pltpu.VMEM((_C,), jnp.float32),
        pltpu.VMEM((_C,), jnp.float32),
        pltpu.VMEM((_C,), jnp.float32),
        pltpu.VMEM((8, _C), jnp.int32),
        pltpu.VMEM((8, _C), jnp.int32),
        pltpu.VMEM((8, _C, 16), jnp.float32),
        pltpu.SemaphoreType.DMA,
    ]
    f = pl.kernel(
        _enc_body,
        out_type=jax.ShapeDtypeStruct((_N_POINTS * 128,), jnp.float32),
        mesh=mesh,
        scratch_types=[
            pltpu.VMEM((3 * _C,), jnp.float32),
            pltpu.VMEM((3 * _C,), jnp.float32),
            pltpu.VMEM((_C * 128,), jnp.float32),
            pltpu.VMEM((_N_LEVELS * 16,), jnp.float32),
            pltpu.VMEM((_N_LEVELS * 16,), jnp.int32),
            pltpu.VMEM((_N_LEVELS * 16,), jnp.int32),
            pltpu.VMEM((_K * 4, 128), jnp.float32),
            pltpu.VMEM((_K * 32, 16), jnp.float32),
            pltpu.HBM((_N_LEVELS * _T // 4, 16), jnp.float32),
        ] + dbuf + dbuf,
        compiler_params=pltpu.CompilerParams(needs_layout_passes=False,
                                             use_tc_tiling_on_sc=False),
    )
    return f(xf, tbl, resf, m1t, m2t)


def _mlp_body(enc_ref, w1_ref, w2_ref, w3_ref, o_ref):
    h = jnp.dot(enc_ref[...], w1_ref[...], preferred_element_type=jnp.float32)
    h = jnp.maximum(h, 0.0)
    h = jnp.dot(h, w2_ref[...], preferred_element_type=jnp.float32)
    h = jnp.maximum(h, 0.0)
    o_ref[...] = jnp.dot(h, w3_ref[...], preferred_element_type=jnp.float32)


@jax.jit
def _mlp(enc, W1, W2, W3):
    bn = 2048
    return pl.pallas_call(
        _mlp_body,
        grid=(_N_POINTS // bn,),
        in_specs=[
            pl.BlockSpec((bn, 128), lambda i: (i, 0)),
            pl.BlockSpec((128, _HIDDEN), lambda i: (0, 0)),
            pl.BlockSpec((_HIDDEN, _HIDDEN), lambda i: (0, 0)),
            pl.BlockSpec((_HIDDEN, _OUT_DIM), lambda i: (0, 0)),
        ],
        out_specs=pl.BlockSpec((bn, _OUT_DIM), lambda i: (i, 0)),
        out_shape=jax.ShapeDtypeStruct((_N_POINTS, _OUT_DIM), jnp.float32),
    )(enc, W1, W2, W3)


def kernel(x, table, W1, W2, W3):
    xf = x.reshape(-1)
    tbl = (table.reshape(_N_LEVELS, _T // 128, 128, _F)
           .swapaxes(2, 3)
           .reshape(_N_LEVELS * _T * _F // 128, 128))
    enc = _encode(xf, tbl, jnp.asarray(_RESF_TAB),
                  jnp.asarray(_M1_TAB), jnp.asarray(_M2_TAB))
    w1p = jnp.pad(W1, ((0, 128 - _D_ENC), (0, 0)))
    return _mlp(enc.reshape(_N_POINTS, 128), w1p, W2, W3)
